# baseline (speedup 1.0000x reference)
import jax
import jax.numpy as jnp
from jax.experimental import pallas as pl

_H, _W, _C = 100, 152, 256
_HP, _WP = _H + 2, _W + 2
_P = _HP * _WP
_S = 1968
_NCHUNK = 8
_Q = _S * _NCHUNK
_GUARD = 160
_N = _GUARD + _Q + 160


def _rpn_head(x_ref, w9_ref, wc_ref, b3_ref, bc_ref, out_ref):
    for c in range(_NCHUNK):
        acc = jnp.zeros((_S, _C), jnp.float32)
        for k in range(9):
            di, dj = divmod(k, 3)
            start = _GUARD + c * _S + (di - 1) * _WP + (dj - 1)
            acc += jnp.dot(x_ref[0, start:start + _S, :], w9_ref[k],
                           preferred_element_type=jnp.float32)
        h = jnp.maximum(acc + b3_ref[0], 0.0)
        out_ref[0, c * _S:(c + 1) * _S, :] = (
            jnp.dot(h, wc_ref[...], preferred_element_type=jnp.float32)
            + bc_ref[0])


def kernel(x, conv3_w, conv3_b, cls_w, cls_b, reg_w, reg_b):
    n = x.shape[0]
    xt = jnp.transpose(x, (0, 2, 3, 1))
    xp = jnp.pad(xt, ((0, 0), (1, 1), (1, 1), (0, 0)))
    xf = jnp.pad(xp.reshape(n, _P, _C),
                 ((0, 0), (_GUARD, _N - _GUARD - _P), (0, 0)))
    w9 = jnp.transpose(conv3_w, (2, 3, 1, 0)).reshape(9, _C, _C)
    wc = jnp.concatenate([reg_w, cls_w], axis=0)[:, :, 0, 0]
    wc = jnp.pad(wc.T, ((0, 0), (0, 1)))
    bc = jnp.pad(jnp.concatenate([reg_b, cls_b]), (0, 1)).reshape(1, 16)
    b3 = conv3_b.reshape(1, _C)

    out = pl.pallas_call(
        _rpn_head,
        grid=(n,),
        in_specs=[
            pl.BlockSpec((1, _N, _C), lambda i: (i, 0, 0)),
            pl.BlockSpec((9, _C, _C), lambda i: (0, 0, 0)),
            pl.BlockSpec((_C, 16), lambda i: (0, 0)),
            pl.BlockSpec((1, _C), lambda i: (0, 0)),
            pl.BlockSpec((1, 16), lambda i: (0, 0)),
        ],
        out_specs=pl.BlockSpec((1, _Q, 16), lambda i: (i, 0, 0)),
        out_shape=jax.ShapeDtypeStruct((n, _Q, 16), jnp.float32),
    )(xf, w9, wc, b3, bc)

    o = out[:, :_P, :].reshape(n, _HP, _WP, 16)[:, 1:_H + 1, 1:_W + 1, :]
    box = o[..., :12].reshape(n, _H * _W * 3, 4)
    cls = o[..., 12:15].reshape(n, _H * _W * 3, 1)
    return (box, cls)


# speedup vs baseline: 1.0752x; 1.0752x over previous
import jax
import jax.numpy as jnp
from jax.experimental import pallas as pl

_H, _W, _C = 100, 152, 256
_HP, _WP = _H + 2, _W + 2
_P = _HP * _WP
_S = 1968
_NCHUNK = 8
_Q = _S * _NCHUNK
_GUARD = 160
_N = _GUARD + _Q + 160


def _rpn_head(x_ref, w9_ref, wc_ref, b3_ref, bc_ref, out_ref):
    for c in range(_NCHUNK):
        acc = jnp.zeros((_S, _C), jnp.float32)
        for k in range(9):
            di, dj = divmod(k, 3)
            start = _GUARD + c * _S + (di - 1) * _WP + (dj - 1)
            acc += jnp.dot(x_ref[0, start:start + _S, :], w9_ref[k],
                           preferred_element_type=jnp.float32)
        h = jnp.maximum(acc + b3_ref[0], 0.0).astype(jnp.bfloat16)
        out_ref[0, c * _S:(c + 1) * _S, :] = (
            jnp.dot(h, wc_ref[...], preferred_element_type=jnp.float32)
            + bc_ref[0])


def kernel(x, conv3_w, conv3_b, cls_w, cls_b, reg_w, reg_b):
    n = x.shape[0]
    xt = jnp.transpose(x, (0, 2, 3, 1))
    xp = jnp.pad(xt, ((0, 0), (1, 1), (1, 1), (0, 0)))
    xf = jnp.pad(xp.reshape(n, _P, _C),
                 ((0, 0), (_GUARD, _N - _GUARD - _P), (0, 0)))
    xf = xf.astype(jnp.bfloat16)
    w9 = jnp.transpose(conv3_w, (2, 3, 1, 0)).reshape(9, _C, _C)
    w9 = w9.astype(jnp.bfloat16)
    wc = jnp.concatenate([reg_w, cls_w], axis=0)[:, :, 0, 0]
    wc = jnp.pad(wc.T, ((0, 0), (0, 1))).astype(jnp.bfloat16)
    bc = jnp.pad(jnp.concatenate([reg_b, cls_b]), (0, 1)).reshape(1, 16)
    b3 = conv3_b.reshape(1, _C)

    out = pl.pallas_call(
        _rpn_head,
        grid=(n,),
        in_specs=[
            pl.BlockSpec((1, _N, _C), lambda i: (i, 0, 0)),
            pl.BlockSpec((9, _C, _C), lambda i: (0, 0, 0)),
            pl.BlockSpec((_C, 16), lambda i: (0, 0)),
            pl.BlockSpec((1, _C), lambda i: (0, 0)),
            pl.BlockSpec((1, 16), lambda i: (0, 0)),
        ],
        out_specs=pl.BlockSpec((1, _Q, 16), lambda i: (i, 0, 0)),
        out_shape=jax.ShapeDtypeStruct((n, _Q, 16), jnp.float32),
    )(xf, w9, wc, b3, bc)

    o = out[:, :_P, :].reshape(n, _HP, _WP, 16)[:, 1:_H + 1, 1:_W + 1, :]
    box = o[..., :12].reshape(n, _H * _W * 3, 4)
    cls = o[..., 12:15].reshape(n, _H * _W * 3, 1)
    return (box, cls)


# speedup vs baseline: 1.2927x; 1.2024x over previous
import jax
import jax.numpy as jnp
from jax.experimental import pallas as pl

_H, _W, _C = 100, 152, 256
_HP, _WP = _H + 2, _W + 2
_P = _HP * _WP
_S = 1968
_NCHUNK = 8
_Q = _S * _NCHUNK
_GUARD = 160
_N = _GUARD + _Q + 160


def _rpn_head(x_ref, w9_ref, wc_ref, b3_ref, bc_ref, out_ref):
    for c in range(_NCHUNK):
        acc = jnp.zeros((_C, _S), jnp.float32)
        for k in range(9):
            di, dj = divmod(k, 3)
            start = _GUARD + c * _S + (di - 1) * _WP + (dj - 1)
            acc += jnp.dot(w9_ref[k], x_ref[0, :, start:start + _S],
                           preferred_element_type=jnp.float32)
        h = jnp.maximum(acc + b3_ref[...], 0.0).astype(jnp.bfloat16)
        out_ref[0, :, c * _S:(c + 1) * _S] = (
            jnp.dot(wc_ref[...], h, preferred_element_type=jnp.float32)
            + bc_ref[...])


def kernel(x, conv3_w, conv3_b, cls_w, cls_b, reg_w, reg_b):
    n = x.shape[0]
    xp = jnp.pad(x, ((0, 0), (0, 0), (1, 1), (1, 1)))
    xf = jnp.pad(xp.reshape(n, _C, _P),
                 ((0, 0), (0, 0), (_GUARD, _N - _GUARD - _P)))
    xf = xf.astype(jnp.bfloat16)
    w9 = jnp.transpose(conv3_w, (2, 3, 0, 1)).reshape(9, _C, _C)
    w9 = w9.astype(jnp.bfloat16)
    wc = jnp.concatenate([reg_w, cls_w], axis=0)[:, :, 0, 0]
    wc = jnp.pad(wc, ((0, 1), (0, 0))).astype(jnp.bfloat16)
    bc = jnp.pad(jnp.concatenate([reg_b, cls_b]), (0, 1)).reshape(16, 1)
    b3 = conv3_b.reshape(_C, 1)

    out = pl.pallas_call(
        _rpn_head,
        grid=(n,),
        in_specs=[
            pl.BlockSpec((1, _C, _N), lambda i: (i, 0, 0)),
            pl.BlockSpec((9, _C, _C), lambda i: (0, 0, 0)),
            pl.BlockSpec((16, _C), lambda i: (0, 0)),
            pl.BlockSpec((_C, 1), lambda i: (0, 0)),
            pl.BlockSpec((16, 1), lambda i: (0, 0)),
        ],
        out_specs=pl.BlockSpec((1, 16, _Q), lambda i: (i, 0, 0)),
        out_shape=jax.ShapeDtypeStruct((n, 16, _Q), jnp.float32),
    )(xf, w9, wc, b3, bc)

    o = out[:, :, :_P].reshape(n, 16, _HP, _WP)[:, :, 1:_H + 1, 1:_W + 1]
    o = jnp.transpose(o, (0, 2, 3, 1))
    box = o[..., :12].reshape(n, _H * _W * 3, 4)
    cls = o[..., 12:15].reshape(n, _H * _W * 3, 1)
    return (box, cls)
